# baseline (speedup 1.0000x reference)
import jax
import jax.numpy as jnp
from jax.experimental import pallas as pl
from jax.experimental.pallas import tpu as pltpu

_W0 = 1.0 / 1223
_W1 = 1.0 / 2444
_W2 = 1.0 / 1687
_GRID = 4
_CHUNK = 16384 // _GRID


def _ce_kernel(x_ref, t_ref, loss_ref, acc_ref):
    i = pl.program_id(0)
    x = x_ref[...]
    t = t_ref[...]
    e = jnp.exp(x)
    lse = jnp.log(e[0:1, :] + e[1:2, :] + e[2:3, :])
    is0 = t == 0
    is1 = t == 1
    picked = jnp.where(is0, x[0:1, :], jnp.where(is1, x[1:2, :], x[2:3, :]))
    w = jnp.where(is0, _W0, jnp.where(is1, _W1, _W2)).astype(jnp.float32)
    num = jnp.sum(w * (lse - picked))
    den = jnp.sum(w)

    @pl.when(i == 0)
    def _():
        acc_ref[0] = num
        acc_ref[1] = den

    @pl.when(i > 0)
    def _():
        acc_ref[0] += num
        acc_ref[1] += den

    @pl.when(i == _GRID - 1)
    def _():
        loss_ref[0, 0] = acc_ref[0] / acc_ref[1]


def kernel(index, output, target, pred_hist):
    del index, pred_hist
    x = output.T
    t = target.reshape(1, 16384)
    loss = pl.pallas_call(
        _ce_kernel,
        grid=(_GRID,),
        in_specs=[
            pl.BlockSpec((3, _CHUNK), lambda i: (0, i)),
            pl.BlockSpec((1, _CHUNK), lambda i: (0, i)),
        ],
        out_shape=jax.ShapeDtypeStruct((1, 1), jnp.float32),
        out_specs=pl.BlockSpec((1, 1), lambda i: (0, 0),
                               memory_space=pltpu.SMEM),
        scratch_shapes=[pltpu.SMEM((2,), jnp.float32)],
    )(x, t)
    return loss[0, 0]
